# baseline (speedup 1.0000x reference)
import functools

import jax
import jax.numpy as jnp
from jax import lax
from jax.experimental import pallas as pl
from jax.experimental.pallas import tpu as pltpu
from jax.experimental.pallas import tpu_sc as plsc

B = 16384
F = 26
V = 100000
C = 13
D = 32
T = F + C

_INFO = plsc.get_sparse_core_info()
NC = _INFO.num_cores
NS = _INFO.num_subcores
NW = NC * NS

ROWS = 1664

CAT_PER_W = (B * F) // NW
CAT_NCHUNK = CAT_PER_W // ROWS

NUM_PER_W = (B * C) // NW
NUM_NCHUNK = NUM_PER_W // ROWS


def _num_body(xn_ref, w_ref, b_ref, o_ref):
    o_ref[...] = (xn_ref[...][:, :, None] * w_ref[...][None, :, :]
                  + b_ref[...][None, :, :])


def _num_tokens(x_num, weight, bias):
    blk = 2048
    return pl.pallas_call(
        _num_body,
        grid=(B // blk,),
        in_specs=[
            pl.BlockSpec((blk, C), lambda i: (i, 0)),
            pl.BlockSpec((C, D), lambda i: (0, 0)),
            pl.BlockSpec((C, D), lambda i: (0, 0)),
        ],
        out_specs=pl.BlockSpec((blk, C, D), lambda i: (i, 0, 0)),
        out_shape=jax.ShapeDtypeStruct((B, C, D), jnp.float32),
    )(x_num, weight, bias)


_sc_mesh = plsc.VectorSubcoreMesh(core_axis_name="c", subcore_axis_name="s")


@functools.partial(
    pl.kernel,
    mesh=_sc_mesh,
    compiler_params=pltpu.CompilerParams(use_tc_tiling_on_sc=False),
    out_type=jax.ShapeDtypeStruct((B * T, D), jnp.float32),
    scratch_types=[
        pltpu.VMEM((ROWS,), jnp.int32),
        pltpu.VMEM((ROWS,), jnp.int32),
        pltpu.VMEM((ROWS, D), jnp.float32),
        pltpu.SemaphoreType.DMA,
        pltpu.SemaphoreType.DMA,
    ],
)
def _sc_scatter(emb_hbm, idxc_hbm, dstc_hbm, num_hbm, dstn_hbm, out_hbm,
                idx_v, dst_v, rows_v, sem_g, sem_s):
    wid = lax.axis_index("s") * NC + lax.axis_index("c")

    for k in range(CAT_NCHUNK):
        r0 = wid * CAT_PER_W + k * ROWS
        pltpu.sync_copy(idxc_hbm.at[pl.ds(r0, ROWS)], idx_v)
        pltpu.sync_copy(dstc_hbm.at[pl.ds(r0, ROWS)], dst_v)
        pltpu.async_copy(emb_hbm.at[idx_v], rows_v, sem_g).wait()
        pltpu.async_copy(rows_v, out_hbm.at[dst_v], sem_s).wait()

    for k in range(NUM_NCHUNK):
        r0 = wid * NUM_PER_W + k * ROWS
        pltpu.sync_copy(dstn_hbm.at[pl.ds(r0, ROWS)], dst_v)
        pltpu.sync_copy(num_hbm.at[pl.ds(r0, ROWS)], rows_v)
        pltpu.async_copy(rows_v, out_hbm.at[dst_v], sem_s).wait()


def kernel(x_cat, x_num, weight, bias, emb_tables):
    emb_flat = emb_tables.reshape(F * V, D)
    idx_cat = (x_cat.astype(jnp.int32)
               + (jnp.arange(F, dtype=jnp.int32) * V)[None, :]).reshape(B * F)
    i = jnp.arange(B * F, dtype=jnp.int32)
    dst_cat = (i // F) * T + (i % F)
    j = jnp.arange(B * C, dtype=jnp.int32)
    dst_num = (j // C) * T + F + (j % C)

    num_tok = _num_tokens(x_num, weight, bias).reshape(B * C, D)
    out = _sc_scatter(emb_flat, idx_cat, dst_cat, num_tok, dst_num)
    return out.reshape(B, T, D)


# speedup vs baseline: 1.0045x; 1.0045x over previous
import functools

import jax
import jax.numpy as jnp
from jax import lax
from jax.experimental import pallas as pl
from jax.experimental.pallas import tpu as pltpu
from jax.experimental.pallas import tpu_sc as plsc

B = 16384
F = 26
V = 100000
C = 13
D = 32
T = F + C

_INFO = plsc.get_sparse_core_info()
NC = _INFO.num_cores
NS = _INFO.num_subcores
NW = NC * NS

BPW = B // NW
NB_CAT = 64
NB_NUM = 16
CAT_NCHUNK = BPW // NB_CAT
NUM_NCHUNK = BPW // NB_NUM
ROWS = NB_CAT * F


def _num_body(xn_ref, w_ref, b_ref, o_ref):
    o_ref[...] = (xn_ref[...][:, :, None] * w_ref[...][None, :, :]
                  + b_ref[...][None, :, :])


def _num_tokens(x_num, weight, bias):
    blk = 2048
    return pl.pallas_call(
        _num_body,
        grid=(B // blk,),
        in_specs=[
            pl.BlockSpec((blk, C), lambda i: (i, 0)),
            pl.BlockSpec((C, D), lambda i: (0, 0)),
            pl.BlockSpec((C, D), lambda i: (0, 0)),
        ],
        out_specs=pl.BlockSpec((blk, C, D), lambda i: (i, 0, 0)),
        out_shape=jax.ShapeDtypeStruct((B, C, D), jnp.float32),
    )(x_num, weight, bias)


_sc_mesh = plsc.VectorSubcoreMesh(core_axis_name="c", subcore_axis_name="s")


@functools.partial(
    pl.kernel,
    mesh=_sc_mesh,
    compiler_params=pltpu.CompilerParams(use_tc_tiling_on_sc=False),
    out_type=jax.ShapeDtypeStruct((B, T, D), jnp.float32),
    scratch_types=[
        pltpu.VMEM((ROWS,), jnp.int32),
        pltpu.VMEM((ROWS,), jnp.int32),
        pltpu.VMEM((ROWS, D), jnp.float32),
        pltpu.VMEM((ROWS, D), jnp.float32),
        pltpu.VMEM((NB_NUM, C, D), jnp.float32),
        pltpu.VMEM((NB_NUM, C, D), jnp.float32),
        pltpu.SemaphoreType.DMA,
        pltpu.SemaphoreType.DMA,
        pltpu.SemaphoreType.DMA,
        pltpu.SemaphoreType.DMA,
        pltpu.SemaphoreType.DMA,
        pltpu.SemaphoreType.DMA,
    ],
)
def _sc_fill(emb_hbm, idxc_hbm, num_hbm, out_hbm,
             idx0, idx1, rows0, rows1, nrow0, nrow1,
             si0, si1, sg0, sg1, ss0, ss1):
    wid = lax.axis_index("s") * NC + lax.axis_index("c")
    b_base = wid * BPW
    cbufs = [(idx0, rows0, si0, sg0, ss0), (idx1, rows1, si1, sg1, ss1)]
    nbufs = [(nrow0, sg0, ss0), (nrow1, sg1, ss1)]

    cat_g = [None] * CAT_NCHUNK
    num_g = [None] * NUM_NCHUNK

    def cat_load(k):
        idx_v, rows_v, si, sg, _ = cbufs[k % 2]
        r0 = wid * (BPW * F) + k * ROWS
        pltpu.async_copy(idxc_hbm.at[pl.ds(r0, ROWS)], idx_v, si).wait()
        cat_g[k] = pltpu.async_copy(emb_hbm.at[idx_v], rows_v, sg)

    def cat_store(k):
        _, rows_v, _, _, ss = cbufs[k % 2]
        b0 = b_base + k * NB_CAT

        def fire(j, carry):
            pltpu.async_copy(rows_v.at[pl.ds(j * F, F)],
                             out_hbm.at[b0 + j, pl.ds(0, F)], ss)
            return carry

        lax.fori_loop(0, NB_CAT, fire, 0)

    def cat_drain(k):
        _, rows_v, _, _, ss = cbufs[k % 2]
        pltpu.make_async_copy(emb_hbm.at[pl.ds(0, ROWS)], rows_v, ss).wait()

    def num_load(k):
        nrow_v, sg, _ = nbufs[k % 2]
        b0 = b_base + k * NB_NUM
        num_g[k] = pltpu.async_copy(num_hbm.at[pl.ds(b0, NB_NUM)], nrow_v, sg)

    cat_load(0)
    cat_load(1)
    for k in range(CAT_NCHUNK):
        cat_g[k].wait()
        cat_store(k)
        cat_drain(k)
        if k + 2 < CAT_NCHUNK:
            cat_load(k + 2)

    num_load(0)
    num_load(1)
    for k in range(NUM_NCHUNK):
        nrow_v, _, ss = nbufs[k % 2]
        b0 = b_base + k * NB_NUM
        num_g[k].wait()
        pltpu.async_copy(nrow_v, out_hbm.at[pl.ds(b0, NB_NUM),
                                            pl.ds(F, C)], ss).wait()
        if k + 2 < NUM_NCHUNK:
            num_load(k + 2)


def kernel(x_cat, x_num, weight, bias, emb_tables):
    emb_flat = emb_tables.reshape(F * V, D)
    idx_cat = (x_cat.astype(jnp.int32)
               + (jnp.arange(F, dtype=jnp.int32) * V)[None, :]).reshape(B * F)
    num_tok = _num_tokens(x_num, weight, bias)
    out = _sc_fill(emb_flat, idx_cat, num_tok)
    return out
